# Initial kernel scaffold; baseline (speedup 1.0000x reference)
import jax
import jax.numpy as jnp
from jax.experimental import pallas as pl


def kernel(status, est_size, degree, edge_index, edge_feat_id, edge_feat_pos, edge_id_tab, edge_pos_tab, Wm1, bm1, Wm2, bm2, Wc1, bc1, Wc2, bc2, Wa1, ba1, Wa2, ba2, Wv1, bv1, Wv2, bv2, Wv3, bv3):
    raise NotImplementedError("write your pallas kernel here")


# speedup vs baseline: 7.5033x; 7.5033x over previous
import functools

import jax
import jax.numpy as jnp
from jax import lax
from jax.experimental import pallas as pl
from jax.experimental.pallas import tpu as pltpu
from jax.experimental.pallas import tpu_sc as plsc

N = 10000
E = 320000
H = 256
VOCAB = 10001
VPAD = 10112
NC, NS, L = 2, 16, 16
K = 64
EPAD = 321536
NPAD = 10240
RPT = NPAD // NS
NBLK = 5
BR = N // NBLK

_f32 = jnp.float32
_i32 = jnp.int32


def _tc1_body(st_ref, es_ref, dg_ref, idp_ref, pos_ref, wm1_ref, bm1_ref,
              wm2_ref, bm2_ref, nfp_ref, wt4_ref):
    i0 = lax.broadcasted_iota(_i32, (80, 128), 0)
    i1 = lax.broadcasted_iota(_i32, (80, 128), 1)
    m = (i0 * 128 + i1) < N

    def _norm(x):
        xm = jnp.where(m, x, 0.0)
        mean = jnp.sum(xm) / N
        d = jnp.where(m, x - mean, 0.0)
        std = jnp.sqrt(jnp.sum(d * d) / (N - 1))
        return jnp.where(std > 1e-8, d / std, d)

    st = st_ref[...]
    es = es_ref[...]
    dg = dg_ref[...]
    anyf = jnp.any(jnp.where(m, dg, 0.0) > 2.0)
    nfp_ref[0] = st
    nfp_ref[1] = _norm(es)
    nfp_ref[2] = jnp.where(anyf, _norm(dg), jnp.ones_like(dg))

    wm1 = wm1_ref[...]
    h1 = jnp.dot(idp_ref[...], wm1[:8, :], preferred_element_type=_f32) \
        + bm1_ref[...]
    p1 = jnp.dot(pos_ref[...], wm1[8:, :], preferred_element_type=_f32)
    wm2 = wm2_ref[...]
    cols = []
    for p in range(3):
        t = jnp.maximum(h1 + p1[p:p + 1, :], 0.0)
        cols.append(jnp.dot(t, wm2, preferred_element_type=_f32))
    cols.append(jnp.zeros_like(cols[0]))
    wt4_ref[...] = jax.nn.sigmoid(jnp.concatenate(cols, axis=1) + bm2_ref[0, 0])


def _tc1(st, es, dg, idp, pos_tab, wm1, bm1, wm2, bm2):
    return pl.pallas_call(
        _tc1_body,
        out_shape=(jax.ShapeDtypeStruct((3, 80, 128), _f32),
                   jax.ShapeDtypeStruct((VPAD, 4), _f32)),
    )(st, es, dg, idp, pos_tab, wm1, bm1, wm2, bm2)


_MESH = plsc.VectorSubcoreMesh(core_axis_name="c", subcore_axis_name="s")
_EPT1 = EPAD // (NC * NS)
_CH1 = _EPT1 // K


@functools.partial(
    pl.kernel,
    out_type=(jax.ShapeDtypeStruct((EPAD,), _f32),
              jax.ShapeDtypeStruct((NC, NPAD), _f32)),
    mesh=_MESH,
    compiler_params=pltpu.CompilerParams(needs_layout_passes=False, use_tc_tiling_on_sc=False),
    scratch_types=[
        pltpu.VMEM((4 * VPAD,), _f32),
        pltpu.VMEM((K,), _i32),
        pltpu.VMEM((K,), _i32),
        pltpu.VMEM((K,), _i32),
        pltpu.VMEM((K,), _f32),
        pltpu.VMEM_SHARED((NPAD,), _f32),
    ])
def _sc1(wt_hbm, ids_hbm, pos_hbm, col_hbm, zer_hbm,
         ew_hbm, degp_hbm, wt_v, id_v, pos_v, col_v, ew_v, deg_sh):
    cid = lax.axis_index("c")
    sid = lax.axis_index("s")
    pltpu.sync_copy(wt_hbm, wt_v)
    pltpu.sync_copy(zer_hbm.at[pl.ds(0, RPT)], deg_sh.at[pl.ds(sid * RPT, RPT)])
    plsc.subcore_barrier()

    base = (cid * NS + sid) * _EPT1

    @pl.loop(0, _CH1)
    def _chunk(ch):
        eb = base + ch * K
        pltpu.sync_copy(ids_hbm.at[pl.ds(eb, K)], id_v)
        pltpu.sync_copy(pos_hbm.at[pl.ds(eb, K)], pos_v)
        pltpu.sync_copy(col_hbm.at[pl.ds(eb, K)], col_v)
        for j in range(K // L):
            idv = id_v[pl.ds(j * L, L)]
            pv = pos_v[pl.ds(j * L, L)]
            flat = idv * 4 + pv
            w16 = plsc.load_gather(wt_v, [flat])
            gpos = eb + j * L + lax.broadcasted_iota(_i32, (L,), 0)
            ew_v[pl.ds(j * L, L)] = jnp.where(gpos < E, w16, 0.0)
        pltpu.sync_copy(ew_v, ew_hbm.at[pl.ds(eb, K)])
        pltpu.sync_copy(ew_v, deg_sh.at[col_v], add=True)

    plsc.subcore_barrier()
    pltpu.sync_copy(deg_sh.at[pl.ds(sid * RPT, RPT)],
                    degp_hbm.at[cid, pl.ds(sid * RPT, RPT)])


def _tc2_body(degp_ref, nfp_ref, dis_ref, vp_ref):
    deg = degp_ref[0] + degp_ref[1] + 1.0
    dis = jax.lax.rsqrt(deg)
    dis_ref[...] = dis
    for k in range(3):
        vp_ref[k] = dis * nfp_ref[k]


def _tc2(degp, nfp):
    return pl.pallas_call(
        _tc2_body,
        out_shape=(jax.ShapeDtypeStruct((80, 128), _f32),
                   jax.ShapeDtypeStruct((3, 80, 128), _f32)),
    )(degp, nfp)


def _make_sc_spmm(W, split32, row_off_by_core, tbl_rows):
    ept = EPAD // (NC * NS) if split32 else EPAD // NS
    chunks = ept // K
    shift = {16: 4, 128: 7}[W]

    @functools.partial(
        pl.kernel,
        out_type=jax.ShapeDtypeStruct((NC, NPAD, W), _f32),
        mesh=_MESH,
        compiler_params=pltpu.CompilerParams(needs_layout_passes=False, use_tc_tiling_on_sc=False),
        scratch_types=[
            pltpu.VMEM((K,), _i32),
            pltpu.VMEM((K,), _i32),
            pltpu.VMEM((K,), _f32),
            pltpu.VMEM((K, W), _f32),
            pltpu.VMEM_SHARED((NPAD, W), _f32),
        ])
    def spmm(tbl_hbm, row_hbm, col_hbm, ew_hbm, zer_hbm,
             out_hbm, row_v, col_v, ew_v, rows_v, acc_sh):
        cid = lax.axis_index("c")
        sid = lax.axis_index("s")
        pltpu.sync_copy(zer_hbm.at[pl.ds(0, RPT)],
                        acc_sh.at[pl.ds(sid * RPT, RPT)])
        plsc.subcore_barrier()

        if split32:
            base = (cid * NS + sid) * ept
        else:
            base = sid * ept

        @pl.loop(0, chunks)
        def _chunk(ch):
            eb = base + ch * K
            pltpu.sync_copy(row_hbm.at[pl.ds(eb, K)], row_v)
            pltpu.sync_copy(col_hbm.at[pl.ds(eb, K)], col_v)
            pltpu.sync_copy(ew_hbm.at[pl.ds(eb, K)], ew_v)
            if row_off_by_core:
                for j in range(K // L):
                    row_v[pl.ds(j * L, L)] = (row_v[pl.ds(j * L, L)]
                                              + cid * tbl_rows)
            pltpu.sync_copy(tbl_hbm.at[row_v], rows_v)

            @pl.loop(0, K)
            def _scale(i):
                s = plsc.load_gather(ew_v, [jnp.full((L,), i, _i32)])
                for j in range(W // L):
                    rows_v[i, pl.ds(j * L, L)] = rows_v[i, pl.ds(j * L, L)] * s

            pltpu.sync_copy(rows_v, acc_sh.at[col_v], add=True)

        plsc.subcore_barrier()
        pltpu.sync_copy(acc_sh.at[pl.ds(sid * RPT, RPT)],
                        out_hbm.at[cid, pl.ds(sid * RPT, RPT)])

    return spmm


_sc2 = _make_sc_spmm(16, split32=True, row_off_by_core=False, tbl_rows=N)
_sc3 = _make_sc_spmm(128, split32=False, row_off_by_core=True, tbl_rows=N)


def _tc3_body(qs0_ref, qs1_ref, v_ref, dis_ref, wc1_ref, bc1_ref,
              u0_ref, u1_ref):
    dis = dis_ref[...]
    q = dis * (qs0_ref[...] + qs1_ref[...] + v_ref[...])
    x1 = jnp.maximum(
        jnp.dot(q, wc1_ref[...], preferred_element_type=_f32) + bc1_ref[...],
        0.0)
    u = dis * x1
    u0_ref[...] = u[:, :H // 2]
    u1_ref[...] = u[:, H // 2:]


def _tc3(qs0, qs1, v16, dis, wc1p, bc1):
    blk = lambda i: (i, 0)
    return pl.pallas_call(
        _tc3_body,
        grid=(NBLK,),
        in_specs=[
            pl.BlockSpec((BR, 16), blk),
            pl.BlockSpec((BR, 16), blk),
            pl.BlockSpec((BR, 16), blk),
            pl.BlockSpec((BR, 1), blk),
            pl.BlockSpec((16, H), lambda i: (0, 0)),
            pl.BlockSpec((1, H), lambda i: (0, 0)),
        ],
        out_specs=(pl.BlockSpec((BR, H // 2), blk),
                   pl.BlockSpec((BR, H // 2), blk)),
        out_shape=(jax.ShapeDtypeStruct((N, H // 2), _f32),
                   jax.ShapeDtypeStruct((N, H // 2), _f32)),
    )(qs0, qs1, v16, dis, wc1p, bc1)


def _tc4_body(a0_ref, a1_ref, u0_ref, u1_ref, dis_ref, w2a_ref, w2b_ref,
              bc2_ref, x2_ref, xsum_ref):
    dis = dis_ref[...]
    y0 = dis * (a0_ref[...] + u0_ref[...])
    y1 = dis * (a1_ref[...] + u1_ref[...])
    x2 = jnp.maximum(
        jnp.dot(y0, w2a_ref[...], preferred_element_type=_f32)
        + jnp.dot(y1, w2b_ref[...], preferred_element_type=_f32)
        + bc2_ref[...], 0.0)
    x2_ref[...] = x2

    @pl.when(pl.program_id(0) == 0)
    def _():
        xsum_ref[...] = jnp.zeros_like(xsum_ref)
    xsum_ref[...] += jnp.sum(x2, axis=0, keepdims=True)


def _tc4(a0, a1, u0, u1, dis, w2a, w2b, bc2):
    blk = lambda i: (i, 0)
    full = lambda i: (0, 0)
    return pl.pallas_call(
        _tc4_body,
        grid=(NBLK,),
        in_specs=[
            pl.BlockSpec((BR, H // 2), blk),
            pl.BlockSpec((BR, H // 2), blk),
            pl.BlockSpec((BR, H // 2), blk),
            pl.BlockSpec((BR, H // 2), blk),
            pl.BlockSpec((BR, 1), blk),
            pl.BlockSpec((H // 2, H), full),
            pl.BlockSpec((H // 2, H), full),
            pl.BlockSpec((1, H), full),
        ],
        out_specs=(pl.BlockSpec((BR, H), blk),
                   pl.BlockSpec((1, H), full)),
        out_shape=(jax.ShapeDtypeStruct((N, H), _f32),
                   jax.ShapeDtypeStruct((1, H), _f32)),
    )(a0, a1, u0, u1, dis, w2a, w2b, bc2)


def _tc5_body(x2_ref, xsum_ref, wa1a_ref, wa1b_ref, ba1_ref, wa2_ref,
              ba2_ref, wv1_ref, bv1_ref, wv2_ref, bv2_ref, wv3_ref, bv3_ref,
              lg_ref, sv_ref):
    gr = xsum_ref[...] * (1.0 / N)
    t = jnp.dot(gr, wa1b_ref[...], preferred_element_type=_f32) \
        + ba1_ref[...]
    hl = jnp.maximum(
        jnp.dot(x2_ref[...], wa1a_ref[...], preferred_element_type=_f32) + t,
        0.0)
    lg_ref[...] = jnp.dot(hl, wa2_ref[...], preferred_element_type=_f32) \
        + ba2_ref[...]

    @pl.when(pl.program_id(0) == 0)
    def _():
        pooled = jnp.concatenate([gr, gr], axis=1)
        v1 = jnp.maximum(
            jnp.dot(pooled, wv1_ref[...], preferred_element_type=_f32)
            + bv1_ref[...], 0.0)
        v2 = jnp.maximum(
            jnp.dot(v1, wv2_ref[...], preferred_element_type=_f32)
            + bv2_ref[...], 0.0)
        sv_ref[...] = jnp.dot(v2, wv3_ref[...], preferred_element_type=_f32) \
            + bv3_ref[...]


def _tc5(x2, xsum, wa1a, wa1b, ba1, wa2, ba2, wv1, bv1, wv2, bv2, wv3, bv3):
    blk = lambda i: (i, 0)
    full = lambda i: (0, 0)
    return pl.pallas_call(
        _tc5_body,
        grid=(NBLK,),
        in_specs=[
            pl.BlockSpec((BR, H), blk),
            pl.BlockSpec((1, H), full),
            pl.BlockSpec((H, H), full),
            pl.BlockSpec((H, H), full),
            pl.BlockSpec((1, H), full),
            pl.BlockSpec((H, 1), full),
            pl.BlockSpec((1, 1), full),
            pl.BlockSpec((2 * H, 2 * H), full),
            pl.BlockSpec((1, 2 * H), full),
            pl.BlockSpec((2 * H, H), full),
            pl.BlockSpec((1, H), full),
            pl.BlockSpec((H, 1), full),
            pl.BlockSpec((1, 1), full),
        ],
        out_specs=(pl.BlockSpec((BR, 1), blk),
                   pl.BlockSpec((1, 1), full)),
        out_shape=(jax.ShapeDtypeStruct((N, 1), _f32),
                   jax.ShapeDtypeStruct((1, 1), _f32)),
    )(x2, xsum, wa1a, wa1b, ba1, wa2, ba2, wv1, bv1, wv2, bv2, wv3, bv3)


def kernel(status, est_size, degree, edge_index, edge_feat_id, edge_feat_pos,
           edge_id_tab, edge_pos_tab, Wm1, bm1, Wm2, bm2,
           Wc1, bc1, Wc2, bc2, Wa1, ba1, Wa2, ba2,
           Wv1, bv1, Wv2, bv2, Wv3, bv3):
    f32 = _f32

    def plane(x):
        return jnp.pad(x.astype(f32), (0, NPAD - N)).reshape(80, 128)

    def epad(x):
        return jnp.pad(x.astype(_i32), (0, EPAD - E))

    st, es, dg = plane(status), plane(est_size), plane(degree)
    idp = jnp.pad(edge_id_tab.astype(f32), ((0, VPAD - VOCAB), (0, 0)))

    nfp, wt4 = _tc1(st, es, dg, idp, edge_pos_tab.astype(f32),
                    Wm1.astype(f32), bm1.astype(f32).reshape(1, 10),
                    Wm2.astype(f32), bm2.astype(f32).reshape(1, 1))

    wt_flat = wt4.reshape(4 * VPAD)
    rows = epad(edge_index[0])
    cols = epad(edge_index[1])
    ids = epad(edge_feat_id)
    poss = epad(edge_feat_pos)
    zeros_rw = jnp.zeros((RPT, 128), f32)

    ew, degp = _sc1(wt_flat, ids, poss, cols, zeros_rw[:, 0])

    dis_p, vp = _tc2(degp.reshape(NC, 80, 128), nfp)

    vT = jnp.transpose(vp.reshape(3, NPAD))[:N]
    v16 = jnp.pad(vT, ((0, 0), (0, 13)))
    dis = dis_p.reshape(NPAD)[:N, None]

    qsum = _sc2(v16, rows, cols, ew, zeros_rw[:, :16])

    wc1p = jnp.pad(Wc1.astype(f32), ((0, 13), (0, 0)))
    u0, u1 = _tc3(qsum[0, :N], qsum[1, :N], v16, dis,
                  wc1p, bc1.astype(f32).reshape(1, H))

    ucat = jnp.concatenate([u0, u1], axis=0)
    acc = _sc3(ucat, rows, cols, ew, zeros_rw)

    x2, xsum = _tc4(acc[0, :N], acc[1, :N], u0, u1, dis,
                    Wc2.astype(f32)[:H // 2], Wc2.astype(f32)[H // 2:],
                    bc2.astype(f32).reshape(1, H))

    lg, sv = _tc5(x2, xsum,
                  Wa1.astype(f32)[:H], Wa1.astype(f32)[H:],
                  ba1.astype(f32).reshape(1, H),
                  Wa2.astype(f32), ba2.astype(f32).reshape(1, 1),
                  Wv1.astype(f32), bv1.astype(f32).reshape(1, 2 * H),
                  Wv2.astype(f32), bv2.astype(f32).reshape(1, H),
                  Wv3.astype(f32), bv3.astype(f32).reshape(1, 1))

    return (lg.reshape(N), sv[0, 0])


# speedup vs baseline: 12.6531x; 1.6863x over previous
import functools

import jax
import jax.numpy as jnp
from jax import lax
from jax.experimental import pallas as pl
from jax.experimental.pallas import tpu as pltpu
from jax.experimental.pallas import tpu_sc as plsc

N = 10000
E = 320000
H = 256
VOCAB = 10001
VPAD = 10112
NC, NS, L = 2, 16, 16
K = 128
EPAD = 327680
NPAD = 10240
RPT = NPAD // NS
NBLK = 5
BR = N // NBLK

_f32 = jnp.float32
_i32 = jnp.int32


def _tc1_body(st_ref, es_ref, dg_ref, idp_ref, pos_ref, wm1_ref, bm1_ref,
              wm2_ref, bm2_ref, nfp_ref, wt4_ref):
    i0 = lax.broadcasted_iota(_i32, (80, 128), 0)
    i1 = lax.broadcasted_iota(_i32, (80, 128), 1)
    m = (i0 * 128 + i1) < N

    def _norm(x):
        xm = jnp.where(m, x, 0.0)
        mean = jnp.sum(xm) / N
        d = jnp.where(m, x - mean, 0.0)
        std = jnp.sqrt(jnp.sum(d * d) / (N - 1))
        return jnp.where(std > 1e-8, d / std, d)

    st = st_ref[...]
    es = es_ref[...]
    dg = dg_ref[...]
    anyf = jnp.any(jnp.where(m, dg, 0.0) > 2.0)
    nfp_ref[0] = st
    nfp_ref[1] = _norm(es)
    nfp_ref[2] = jnp.where(anyf, _norm(dg), jnp.ones_like(dg))

    wm1 = wm1_ref[...]
    h1 = jnp.dot(idp_ref[...], wm1[:8, :], preferred_element_type=_f32) \
        + bm1_ref[...]
    p1 = jnp.dot(pos_ref[...], wm1[8:, :], preferred_element_type=_f32)
    wm2 = wm2_ref[...]
    cols = []
    for p in range(3):
        t = jnp.maximum(h1 + p1[p:p + 1, :], 0.0)
        cols.append(jnp.dot(t, wm2, preferred_element_type=_f32))
    cols.append(jnp.zeros_like(cols[0]))
    wt4_ref[...] = jax.nn.sigmoid(jnp.concatenate(cols, axis=1) + bm2_ref[0, 0])


def _tc1(st, es, dg, idp, pos_tab, wm1, bm1, wm2, bm2):
    return pl.pallas_call(
        _tc1_body,
        out_shape=(jax.ShapeDtypeStruct((3, 80, 128), _f32),
                   jax.ShapeDtypeStruct((VPAD, 4), _f32)),
    )(st, es, dg, idp, pos_tab, wm1, bm1, wm2, bm2)


_MESH = plsc.VectorSubcoreMesh(core_axis_name="c", subcore_axis_name="s")
_EPT1 = EPAD // (NC * NS)
_CH1 = _EPT1 // K


@functools.partial(
    pl.kernel,
    out_type=(jax.ShapeDtypeStruct((EPAD,), _f32),
              jax.ShapeDtypeStruct((NC, NPAD), _f32)),
    mesh=_MESH,
    compiler_params=pltpu.CompilerParams(needs_layout_passes=False, use_tc_tiling_on_sc=False),
    scratch_types=[
        pltpu.VMEM((4 * VPAD,), _f32),
        pltpu.VMEM((_EPT1,), _i32),
        pltpu.VMEM((_EPT1,), _i32),
        pltpu.VMEM((_CH1, K), _i32),
        pltpu.VMEM((_EPT1,), _f32),
        pltpu.VMEM_SHARED((NPAD,), _f32),
        pltpu.SemaphoreType.DMA,
    ])
def _sc1(wt_hbm, ids_hbm, pos_hbm, col2_hbm, zer_hbm,
         ew_hbm, degp_hbm, wt_v, id_v, pos_v, col_v, ew_v, deg_sh, dsem):
    cid = lax.axis_index("c")
    sid = lax.axis_index("s")
    gbase = (cid * NS + sid) * _EPT1
    cbase = (cid * NS + sid) * _CH1
    pltpu.sync_copy(wt_hbm, wt_v)
    pltpu.sync_copy(ids_hbm.at[pl.ds(gbase, _EPT1)], id_v)
    pltpu.sync_copy(pos_hbm.at[pl.ds(gbase, _EPT1)], pos_v)
    pltpu.sync_copy(col2_hbm.at[pl.ds(cbase, _CH1)], col_v)
    pltpu.sync_copy(zer_hbm.at[pl.ds(0, RPT)], deg_sh.at[pl.ds(sid * RPT, RPT)])
    plsc.subcore_barrier()

    @pl.loop(0, _EPT1 // L, unroll=4)
    def _ew(i):
        flat = id_v[pl.ds(i * L, L)] * 4 + pos_v[pl.ds(i * L, L)]
        w16 = plsc.load_gather(wt_v, [flat])
        gpos = gbase + i * L + lax.broadcasted_iota(_i32, (L,), 0)
        ew_v[pl.ds(i * L, L)] = jnp.where(gpos < E, w16, 0.0)

    pltpu.sync_copy(ew_v, ew_hbm.at[pl.ds(gbase, _EPT1)])

    @pl.loop(0, _CH1)
    def _fire(c):
        pltpu.async_copy(ew_v.at[pl.ds(c * K, K)], deg_sh.at[col_v.at[c]],
                         dsem, add=True)

    @pl.loop(0, _CH1)
    def _drain(c):
        pltpu.make_async_copy(ew_v.at[pl.ds(c * K, K)],
                              deg_sh.at[col_v.at[c]], dsem).wait()

    plsc.subcore_barrier()
    pltpu.sync_copy(deg_sh.at[pl.ds(sid * RPT, RPT)],
                    degp_hbm.at[cid, pl.ds(sid * RPT, RPT)])


def _tc2_body(degp_ref, nfp_ref, dis_ref, vp_ref):
    deg = degp_ref[0] + degp_ref[1] + 1.0
    dis = jax.lax.rsqrt(deg)
    dis_ref[...] = dis
    for k in range(3):
        vp_ref[k] = dis * nfp_ref[k]


def _tc2(degp, nfp):
    return pl.pallas_call(
        _tc2_body,
        out_shape=(jax.ShapeDtypeStruct((80, 128), _f32),
                   jax.ShapeDtypeStruct((3, 80, 128), _f32)),
    )(degp, nfp)


def _make_sc_spmm(W, split32, row_off_by_core, tbl_rows, passes):
    ept = EPAD // (NC * NS) if split32 else EPAD // NS
    nch = ept // K

    @functools.partial(
        pl.kernel,
        out_type=jax.ShapeDtypeStruct((passes, NC, NPAD, W), _f32),
        mesh=_MESH,
        compiler_params=pltpu.CompilerParams(needs_layout_passes=False, use_tc_tiling_on_sc=False),
        scratch_types=[
            pltpu.VMEM((ept,), _i32),
            pltpu.VMEM((nch, K), _i32),
            pltpu.VMEM((ept,), _f32),
            pltpu.VMEM((K, W), _f32),
            pltpu.VMEM((K, W), _f32),
            pltpu.VMEM((K,), _i32),
            pltpu.VMEM((K,), _i32),
            pltpu.VMEM_SHARED((NPAD, W), _f32),
            pltpu.SemaphoreType.DMA,
            pltpu.SemaphoreType.DMA,
            pltpu.SemaphoreType.DMA,
            pltpu.SemaphoreType.DMA,
        ])
    def spmm(tbl_hbm, row_hbm, col2_hbm, ew_hbm, zer_hbm,
             out_hbm, row_v, col_v, ew_v, bufa, bufb, adja, adjb, acc_sh,
             gsa, gsb, ssa, ssb):
        cid = lax.axis_index("c")
        sid = lax.axis_index("s")
        if split32:
            gbase = (cid * NS + sid) * ept
        else:
            gbase = sid * ept
        cbase = gbase // K
        pltpu.sync_copy(row_hbm.at[pl.ds(gbase, ept)], row_v)
        pltpu.sync_copy(col2_hbm.at[pl.ds(cbase, nch)], col_v)
        pltpu.sync_copy(ew_hbm.at[pl.ds(gbase, ept)], ew_v)

        bufs = (bufa, bufb)
        adjs = (adja, adjb)
        gs = (gsa, gsb)
        ss = (ssa, ssb)

        for g in range(passes):
            if row_off_by_core:
                off = (cid * passes + g) * tbl_rows
            else:
                off = g * tbl_rows

            def gstart(b, lc):
                if passes > 1 or row_off_by_core:
                    for j in range(K // L):
                        adjs[b][pl.ds(j * L, L)] = \
                            row_v[pl.ds(lc * K + j * L, L)] + off
                    idx = adjs[b]
                else:
                    idx = row_v.at[pl.ds(lc * K, K)]
                pltpu.async_copy(tbl_hbm.at[idx], bufs[b], gs[b])

            def gwait(b):
                pltpu.make_async_copy(tbl_hbm.at[adjs[b]], bufs[b],
                                      gs[b]).wait()

            def sstart(b, lc):
                pltpu.async_copy(bufs[b], acc_sh.at[col_v.at[lc]], ss[b],
                                 add=True)

            def swait(b, lc):
                pltpu.make_async_copy(bufs[b], acc_sh.at[col_v.at[lc]],
                                      ss[b]).wait()

            def scale(b, lc):
                ebase = lc * K

                @pl.loop(0, K, unroll=4)
                def _row(i):
                    s = plsc.load_gather(ew_v,
                                         [jnp.full((L,), ebase + i, _i32)])
                    for j in range(W // L):
                        bufs[b][i, pl.ds(j * L, L)] = \
                            bufs[b][i, pl.ds(j * L, L)] * s

            pltpu.sync_copy(zer_hbm.at[pl.ds(0, RPT)],
                            acc_sh.at[pl.ds(sid * RPT, RPT)])
            plsc.subcore_barrier()

            gstart(0, 0)

            @pl.loop(0, nch // 2)
            def _pair(t):
                c0 = 2 * t
                gstart(1, c0 + 1)
                gwait(0)
                scale(0, c0)
                sstart(0, c0)
                gwait(1)
                scale(1, c0 + 1)
                sstart(1, c0 + 1)
                swait(0, c0)

                @pl.when(t + 1 < nch // 2)
                def _():
                    gstart(0, c0 + 2)

                swait(1, c0 + 1)

            plsc.subcore_barrier()
            pltpu.sync_copy(acc_sh.at[pl.ds(sid * RPT, RPT)],
                            out_hbm.at[g, cid, pl.ds(sid * RPT, RPT)])

    return spmm


_sc2 = _make_sc_spmm(16, split32=True, row_off_by_core=False, tbl_rows=N, passes=1)
_sc3 = _make_sc_spmm(64, split32=False, row_off_by_core=True, tbl_rows=N, passes=2)


def _tc3_body(qs0_ref, qs1_ref, v_ref, dis_ref, wc1_ref, bc1_ref,
              u0_ref, u1_ref):
    dis = dis_ref[...]
    q = dis * (qs0_ref[...] + qs1_ref[...] + v_ref[...])
    x1 = jnp.maximum(
        jnp.dot(q, wc1_ref[...], preferred_element_type=_f32) + bc1_ref[...],
        0.0)
    u = dis * x1
    u0_ref[...] = u[:, :H // 2]
    u1_ref[...] = u[:, H // 2:]


def _tc3(qs0, qs1, v16, dis, wc1p, bc1):
    blk = lambda i: (i, 0)
    return pl.pallas_call(
        _tc3_body,
        grid=(NBLK,),
        in_specs=[
            pl.BlockSpec((BR, 16), blk),
            pl.BlockSpec((BR, 16), blk),
            pl.BlockSpec((BR, 16), blk),
            pl.BlockSpec((BR, 1), blk),
            pl.BlockSpec((16, H), lambda i: (0, 0)),
            pl.BlockSpec((1, H), lambda i: (0, 0)),
        ],
        out_specs=(pl.BlockSpec((BR, H // 2), blk),
                   pl.BlockSpec((BR, H // 2), blk)),
        out_shape=(jax.ShapeDtypeStruct((N, H // 2), _f32),
                   jax.ShapeDtypeStruct((N, H // 2), _f32)),
    )(qs0, qs1, v16, dis, wc1p, bc1)


def _tc4_body(a00_ref, a01_ref, a10_ref, a11_ref, u0_ref, u1_ref, dis_ref,
              w2a_ref, w2b_ref, bc2_ref, x2_ref, xsum_ref):
    dis = dis_ref[...]
    y0 = dis * (jnp.concatenate([a00_ref[...], a01_ref[...]], axis=1)
                + u0_ref[...])
    y1 = dis * (jnp.concatenate([a10_ref[...], a11_ref[...]], axis=1)
                + u1_ref[...])
    x2 = jnp.maximum(
        jnp.dot(y0, w2a_ref[...], preferred_element_type=_f32)
        + jnp.dot(y1, w2b_ref[...], preferred_element_type=_f32)
        + bc2_ref[...], 0.0)
    x2_ref[...] = x2

    @pl.when(pl.program_id(0) == 0)
    def _():
        xsum_ref[...] = jnp.zeros_like(xsum_ref)
    xsum_ref[...] += jnp.sum(x2, axis=0, keepdims=True)


def _tc4(a00, a01, a10, a11, u0, u1, dis, w2a, w2b, bc2):
    blk = lambda i: (i, 0)
    full = lambda i: (0, 0)
    return pl.pallas_call(
        _tc4_body,
        grid=(NBLK,),
        in_specs=[
            pl.BlockSpec((BR, H // 4), blk),
            pl.BlockSpec((BR, H // 4), blk),
            pl.BlockSpec((BR, H // 4), blk),
            pl.BlockSpec((BR, H // 4), blk),
            pl.BlockSpec((BR, H // 2), blk),
            pl.BlockSpec((BR, H // 2), blk),
            pl.BlockSpec((BR, 1), blk),
            pl.BlockSpec((H // 2, H), full),
            pl.BlockSpec((H // 2, H), full),
            pl.BlockSpec((1, H), full),
        ],
        out_specs=(pl.BlockSpec((BR, H), blk),
                   pl.BlockSpec((1, H), full)),
        out_shape=(jax.ShapeDtypeStruct((N, H), _f32),
                   jax.ShapeDtypeStruct((1, H), _f32)),
    )(a00, a01, a10, a11, u0, u1, dis, w2a, w2b, bc2)


def _tc5_body(x2_ref, xsum_ref, wa1a_ref, wa1b_ref, ba1_ref, wa2_ref,
              ba2_ref, wv1_ref, bv1_ref, wv2_ref, bv2_ref, wv3_ref, bv3_ref,
              lg_ref, sv_ref):
    gr = xsum_ref[...] * (1.0 / N)
    t = jnp.dot(gr, wa1b_ref[...], preferred_element_type=_f32) \
        + ba1_ref[...]
    hl = jnp.maximum(
        jnp.dot(x2_ref[...], wa1a_ref[...], preferred_element_type=_f32) + t,
        0.0)
    lg_ref[...] = jnp.dot(hl, wa2_ref[...], preferred_element_type=_f32) \
        + ba2_ref[...]

    @pl.when(pl.program_id(0) == 0)
    def _():
        pooled = jnp.concatenate([gr, gr], axis=1)
        v1 = jnp.maximum(
            jnp.dot(pooled, wv1_ref[...], preferred_element_type=_f32)
            + bv1_ref[...], 0.0)
        v2 = jnp.maximum(
            jnp.dot(v1, wv2_ref[...], preferred_element_type=_f32)
            + bv2_ref[...], 0.0)
        sv_ref[...] = jnp.dot(v2, wv3_ref[...], preferred_element_type=_f32) \
            + bv3_ref[...]


def _tc5(x2, xsum, wa1a, wa1b, ba1, wa2, ba2, wv1, bv1, wv2, bv2, wv3, bv3):
    blk = lambda i: (i, 0)
    full = lambda i: (0, 0)
    return pl.pallas_call(
        _tc5_body,
        grid=(NBLK,),
        in_specs=[
            pl.BlockSpec((BR, H), blk),
            pl.BlockSpec((1, H), full),
            pl.BlockSpec((H, H), full),
            pl.BlockSpec((H, H), full),
            pl.BlockSpec((1, H), full),
            pl.BlockSpec((H, 1), full),
            pl.BlockSpec((1, 1), full),
            pl.BlockSpec((2 * H, 2 * H), full),
            pl.BlockSpec((1, 2 * H), full),
            pl.BlockSpec((2 * H, H), full),
            pl.BlockSpec((1, H), full),
            pl.BlockSpec((H, 1), full),
            pl.BlockSpec((1, 1), full),
        ],
        out_specs=(pl.BlockSpec((BR, 1), blk),
                   pl.BlockSpec((1, 1), full)),
        out_shape=(jax.ShapeDtypeStruct((N, 1), _f32),
                   jax.ShapeDtypeStruct((1, 1), _f32)),
    )(x2, xsum, wa1a, wa1b, ba1, wa2, ba2, wv1, bv1, wv2, bv2, wv3, bv3)


def kernel(status, est_size, degree, edge_index, edge_feat_id, edge_feat_pos,
           edge_id_tab, edge_pos_tab, Wm1, bm1, Wm2, bm2,
           Wc1, bc1, Wc2, bc2, Wa1, ba1, Wa2, ba2,
           Wv1, bv1, Wv2, bv2, Wv3, bv3):
    f32 = _f32

    def plane(x):
        return jnp.pad(x.astype(f32), (0, NPAD - N)).reshape(80, 128)

    def epad(x):
        return jnp.pad(x.astype(_i32), (0, EPAD - E))

    st, es, dg = plane(status), plane(est_size), plane(degree)
    idp = jnp.pad(edge_id_tab.astype(f32), ((0, VPAD - VOCAB), (0, 0)))

    nfp, wt4 = _tc1(st, es, dg, idp, edge_pos_tab.astype(f32),
                    Wm1.astype(f32), bm1.astype(f32).reshape(1, 10),
                    Wm2.astype(f32), bm2.astype(f32).reshape(1, 1))

    wt_flat = wt4.reshape(4 * VPAD)
    rows = epad(edge_index[0])
    cols2 = epad(edge_index[1]).reshape(EPAD // K, K)
    ids = epad(edge_feat_id)
    poss = epad(edge_feat_pos)
    zeros_rw = jnp.zeros((RPT, 128), f32)

    ew, degp = _sc1(wt_flat, ids, poss, cols2, zeros_rw[:, 0])

    dis_p, vp = _tc2(degp.reshape(NC, 80, 128), nfp)

    vT = jnp.transpose(vp.reshape(3, NPAD))[:N]
    v16 = jnp.pad(vT, ((0, 0), (0, 13)))
    dis = dis_p.reshape(NPAD)[:N, None]

    qsum = _sc2(v16, rows, cols2, ew, zeros_rw[:, :16])[0]

    wc1p = jnp.pad(Wc1.astype(f32), ((0, 13), (0, 0)))
    u0, u1 = _tc3(qsum[0, :N], qsum[1, :N], v16, dis,
                  wc1p, bc1.astype(f32).reshape(1, H))

    ucat = jnp.concatenate([u0[:, :64], u0[:, 64:],
                            u1[:, :64], u1[:, 64:]], axis=0)
    acc = _sc3(ucat, rows, cols2, ew, zeros_rw[:, :64])

    x2, xsum = _tc4(acc[0, 0, :N], acc[1, 0, :N], acc[0, 1, :N], acc[1, 1, :N],
                    u0, u1, dis,
                    Wc2.astype(f32)[:H // 2], Wc2.astype(f32)[H // 2:],
                    bc2.astype(f32).reshape(1, H))

    lg, sv = _tc5(x2, xsum,
                  Wa1.astype(f32)[:H], Wa1.astype(f32)[H:],
                  ba1.astype(f32).reshape(1, H),
                  Wa2.astype(f32), ba2.astype(f32).reshape(1, 1),
                  Wv1.astype(f32), bv1.astype(f32).reshape(1, 2 * H),
                  Wv2.astype(f32), bv2.astype(f32).reshape(1, H),
                  Wv3.astype(f32), bv3.astype(f32).reshape(1, 1))

    return (lg.reshape(N), sv[0, 0])
